# baseline (speedup 1.0000x reference)
import functools

import jax
import jax.numpy as jnp
from jax import lax
from jax.experimental import pallas as pl
from jax.experimental.pallas import tpu as pltpu
from jax.experimental.pallas import tpu_sc as plsc

N_FEAT = 26
BATCH = 16384
VOCAB = 10
DIM = 3

NC = 2
NS = 16
NW = NC * NS
BPW = BATCH // NW
L = 16
NVEC = BPW // L

_mesh = plsc.VectorSubcoreMesh(core_axis_name="c", subcore_axis_name="s")


@functools.partial(
    pl.kernel,
    out_type=jax.ShapeDtypeStruct((BATCH, DIM), jnp.float32),
    mesh=_mesh,
    compiler_params=pltpu.CompilerParams(needs_layout_passes=False),
    scratch_types=[
        pltpu.VMEM((N_FEAT, BPW), jnp.int32),
        pltpu.VMEM((N_FEAT * VOCAB * 2,), jnp.int32),
        pltpu.VMEM((BPW, DIM), jnp.float32),
        pltpu.SemaphoreType.DMA,
    ],
)
def _merger(*refs):
    feats = refs[:N_FEAT]
    tab_hbm = refs[N_FEAT]
    out_hbm = refs[N_FEAT + 1]
    feat_v, tab_v, out_v, sem = refs[N_FEAT + 2:]

    wid = lax.axis_index("s") * NC + lax.axis_index("c")
    base = wid * BPW

    copies = []
    for i in range(N_FEAT):
        copies.append(
            pltpu.make_async_copy(feats[i].at[pl.ds(base, BPW)], feat_v.at[i], sem)
        )
    copies.append(pltpu.make_async_copy(tab_hbm, tab_v, sem))
    for c in copies:
        c.start()
    for c in copies:
        c.wait()

    def body(j, carry):
        col = j * L
        acc01 = jnp.zeros((2 * L,), jnp.bfloat16)
        acc2 = jnp.zeros((L,), jnp.float32)
        for i in range(N_FEAT):
            f = feat_v[i, pl.ds(col, L)]
            p2 = f + f + (i * VOCAB * 2)
            g01 = plsc.load_gather(tab_v, [p2])
            g2 = plsc.load_gather(tab_v, [p2 + 1])
            acc01 = acc01 + plsc.bitcast(g01, jnp.bfloat16)
            acc2 = acc2 + plsc.bitcast(g2, jnp.float32)
        d0, d1 = plsc.unpack(acc01, format=plsc.PackFormat.INTERLEAVED)
        rows = col + lax.iota(jnp.int32, L)
        plsc.store_scatter(
            out_v, [rows, jnp.full((L,), 0, jnp.int32)], d0.astype(jnp.float32)
        )
        plsc.store_scatter(
            out_v, [rows, jnp.full((L,), 1, jnp.int32)], d1.astype(jnp.float32)
        )
        plsc.store_scatter(out_v, [rows, jnp.full((L,), 2, jnp.int32)], acc2)
        return carry

    lax.fori_loop(0, NVEC, body, 0)
    pltpu.sync_copy(out_v, out_hbm.at[pl.ds(base, BPW)])


def kernel(*args):
    feats = args[:N_FEAT]
    tabs = args[N_FEAT:2 * N_FEAT]
    t = jnp.stack(tabs)
    b01 = t[..., :2].astype(jnp.bfloat16)
    bits0 = lax.bitcast_convert_type(b01[..., 0], jnp.uint16).astype(jnp.uint32)
    bits1 = lax.bitcast_convert_type(b01[..., 1], jnp.uint16).astype(jnp.uint32)
    w01 = lax.bitcast_convert_type(bits0 | (bits1 << 16), jnp.int32)
    w2 = lax.bitcast_convert_type(t[..., 2], jnp.int32)
    words = jnp.stack([w01, w2], axis=-1).reshape(-1)
    return _merger(*feats, words)


# speedup vs baseline: 1.0495x; 1.0495x over previous
import functools

import jax
import jax.numpy as jnp
from jax import lax
from jax.experimental import pallas as pl
from jax.experimental.pallas import tpu as pltpu
from jax.experimental.pallas import tpu_sc as plsc

N_FEAT = 26
BATCH = 16384
VOCAB = 10
DIM = 3

NC = 2
NS = 16
NW = NC * NS
BPW = BATCH // NW
L = 16
NVEC = BPW // L

_mesh = plsc.VectorSubcoreMesh(core_axis_name="c", subcore_axis_name="s")


@functools.partial(
    pl.kernel,
    out_type=jax.ShapeDtypeStruct((BATCH, DIM), jnp.float32),
    mesh=_mesh,
    compiler_params=pltpu.CompilerParams(needs_layout_passes=False),
    scratch_types=[
        pltpu.VMEM((N_FEAT, BPW), jnp.int32),
        pltpu.VMEM((N_FEAT * VOCAB * DIM,), jnp.float32),
        pltpu.VMEM((BPW, DIM), jnp.float32),
        pltpu.SemaphoreType.DMA,
    ],
)
def _merger(*refs):
    feats = refs[:N_FEAT]
    tab_hbm = refs[N_FEAT]
    out_hbm = refs[N_FEAT + 1]
    feat_v, tab_v, out_v, sem = refs[N_FEAT + 2:]

    wid = lax.axis_index("s") * NC + lax.axis_index("c")
    base = wid * BPW

    copies = []
    for i in range(N_FEAT):
        copies.append(
            pltpu.make_async_copy(feats[i].at[pl.ds(base, BPW)], feat_v.at[i], sem)
        )
    copies.append(pltpu.make_async_copy(tab_hbm, tab_v, sem))
    for c in copies:
        c.start()
    for c in copies:
        c.wait()

    def body(j, carry):
        col = j * L
        acc = [jnp.zeros((L,), jnp.float32) for _ in range(DIM)]
        for i in range(N_FEAT):
            f3 = feat_v[i, pl.ds(col, L)] * 3
            for d in range(DIM):
                idx = f3 + (i * VOCAB * DIM + d)
                acc[d] = acc[d] + plsc.load_gather(tab_v, [idx])
        rows = col + lax.iota(jnp.int32, L)
        for d in range(DIM):
            plsc.store_scatter(out_v, [rows, jnp.full((L,), d, jnp.int32)], acc[d])
        return carry

    lax.fori_loop(0, NVEC, body, 0)
    pltpu.sync_copy(out_v, out_hbm.at[pl.ds(base, BPW)])


def kernel(*args):
    feats = args[:N_FEAT]
    tabs = args[N_FEAT:2 * N_FEAT]
    tab_flat = jnp.stack(tabs).reshape(-1)
    return _merger(*feats, tab_flat)
